# baseline (speedup 1.0000x reference)
import jax
import jax.numpy as jnp
from jax.experimental import pallas as pl
from jax.experimental.pallas import tpu as pltpu

_U_ROWS, _I_ROWS, _DIM = 100000, 1000000, 32
_GRID = 16
_U_W = 6400
_I_W = 64000


def _copy_body(u_in, i_in, u_out, i_out):
    u_out[...] = u_in[...]
    i_out[...] = i_in[...]


def kernel(user_weight, item_weight, edge_index):
    u_t = user_weight.T
    i_t = item_weight.T
    out_shape = (
        jax.ShapeDtypeStruct(u_t.shape, u_t.dtype),
        jax.ShapeDtypeStruct(i_t.shape, i_t.dtype),
    )
    uo, io = pl.pallas_call(
        _copy_body,
        grid=(_GRID,),
        in_specs=[
            pl.BlockSpec((_DIM, _U_W), lambda g: (0, g)),
            pl.BlockSpec((_DIM, _I_W), lambda g: (0, g)),
        ],
        out_specs=(
            pl.BlockSpec((_DIM, _U_W), lambda g: (0, g)),
            pl.BlockSpec((_DIM, _I_W), lambda g: (0, g)),
        ),
        out_shape=out_shape,
        compiler_params=pltpu.CompilerParams(
            dimension_semantics=("arbitrary",),
        ),
    )(u_t, i_t)
    return uo.T, io.T
